# baseline (speedup 1.0000x reference)
import functools

import jax
import jax.numpy as jnp
from jax import lax
from jax.experimental import pallas as pl
from jax.experimental.pallas import tpu as pltpu
from jax.experimental.pallas import tpu_sc as plsc

_N_NODES = 10000
_N_EDGES = 320000
_PROJ_DIM = 128
_GAT_DIM = 64

_NC = 2
_NS = 16
_NW = _NC * _NS
_L = 16
_CHUNK = _N_EDGES // _NW
_UNROLL = 5
_EWIN = _CHUNK + 112

_HI = -65536

_DN_T = (((1,), (1,)), ((), ()))


def _pack_bf16_pair(p, q):
    pi = lax.bitcast_convert_type(p, jnp.int32)
    qi = lax.bitcast_convert_type(q, jnp.int32)
    hi = pi & _HI
    lo = lax.shift_right_logical(qi + 0x8000, 16)
    return hi | lo


def _table_body(zb_ref, zp_ref, A_ref, Wb_ref, Wp_ref, g_ref, ob_ref, op_ref):
    A = A_ref[...]
    Vb = jnp.dot(A[0:2, :], Wb_ref[...], preferred_element_type=jnp.float32)
    Vp = jnp.dot(A[2:4, :], Wp_ref[...], preferred_element_type=jnp.float32)
    Tb = jnp.transpose(lax.dot_general(zb_ref[...], Vb, _DN_T,
                                       preferred_element_type=jnp.float32))
    Tp = jnp.transpose(lax.dot_general(zp_ref[...], Vp, _DN_T,
                                       preferred_element_type=jnp.float32))
    e0 = jnp.exp(g_ref[0, 0])
    e1 = jnp.exp(g_ref[0, 1])
    w0 = e0 / (e0 + e1)
    w1 = e1 / (e0 + e1)
    ob_ref[...] = _pack_bf16_pair(Tb[0:1, :] * w0, Tb[1:2, :] * w0)
    op_ref[...] = _pack_bf16_pair(Tp[0:1, :] * w1, Tp[1:2, :] * w1)


_table_call = pl.pallas_call(
    _table_body,
    out_shape=(jax.ShapeDtypeStruct((1, _N_NODES), jnp.int32),
               jax.ShapeDtypeStruct((1, _N_NODES), jnp.int32)),
)


def _unpack(w):
    p = plsc.bitcast(w, jnp.float32)
    q = plsc.bitcast(lax.shift_left(w, 16), jnp.float32)
    return p, q


def _edge_body(tb_hbm, tp_hbm, ei_hbm, out_hbm, tb_v, tp_v, ed_v, out_v):
    wid = lax.axis_index("s") * _NC + lax.axis_index("c")
    base = wid * _CHUNK
    astart = pl.multiple_of(base - (base % 128), 128)
    pad = base % 128

    pltpu.sync_copy(tb_hbm.at[0], tb_v)
    pltpu.sync_copy(tp_hbm.at[0], tp_v)
    pltpu.sync_copy(ei_hbm.at[:, pl.ds(astart, _EWIN)], ed_v)

    @plsc.parallel_loop(0, _CHUNK, _L, unroll=_UNROLL)
    def step(off):
        s = ed_v[0, pl.ds(pad + off, _L)]
        d = ed_v[1, pl.ds(pad + off, _L)]
        pb_s, qb_s = _unpack(plsc.load_gather(tb_v, [s]))
        pb_d, qb_d = _unpack(plsc.load_gather(tb_v, [d]))
        pp_s, qp_s = _unpack(plsc.load_gather(tp_v, [s]))
        pp_d, qp_d = _unpack(plsc.load_gather(tp_v, [d]))
        x1b = pb_s + qb_d
        x2b = pb_d + qb_s
        x1p = pp_s + qp_d
        x2p = pp_d + qp_s
        lin = (x1b + x2b) + (x1p + x2p)
        mag = (jnp.abs(x1b) + jnp.abs(x2b)) + (jnp.abs(x1p) + jnp.abs(x2p))
        sc = 0.3 * lin + 0.2 * mag
        out_v[pl.ds(off, _L)] = 1.0 / (1.0 + jnp.exp(-sc))

    pltpu.sync_copy(out_v, out_hbm.at[pl.ds(base, _CHUNK)])


@functools.cache
def _edge_call():
  return functools.partial(
    pl.kernel,
    mesh=plsc.VectorSubcoreMesh(core_axis_name="c", subcore_axis_name="s"),
    out_type=jax.ShapeDtypeStruct((_N_EDGES,), jnp.float32),
    scratch_types=[
        pltpu.VMEM((_N_NODES,), jnp.int32),
        pltpu.VMEM((_N_NODES,), jnp.int32),
        pltpu.VMEM((2, _EWIN), jnp.int32),
        pltpu.VMEM((_CHUNK,), jnp.float32),
    ],
    compiler_params=pltpu.CompilerParams(needs_layout_passes=False),
  )(_edge_body)


def kernel(z_beh, z_pref, edge_index_undirected, W_b, W_p, a_b, a_p,
           gamma_b, gamma_p):
    A = jnp.stack([a_b[:_GAT_DIM], a_b[_GAT_DIM:],
                   a_p[:_GAT_DIM], a_p[_GAT_DIM:]], axis=0)
    g = jnp.stack([gamma_b, gamma_p]).reshape(1, 2).astype(jnp.float32)
    tb, tp = _table_call(z_beh, z_pref, A, W_b, W_p, g)
    return _edge_call()(tb, tp, edge_index_undirected)


# speedup vs baseline: 1.0996x; 1.0996x over previous
import functools

import jax
import jax.numpy as jnp
from jax import lax
from jax.experimental import pallas as pl
from jax.experimental.pallas import tpu as pltpu
from jax.experimental.pallas import tpu_sc as plsc

_N_NODES = 10000
_N_EDGES = 320000
_PROJ_DIM = 128
_GAT_DIM = 64

_NC = 2
_NS = 16
_NW = _NC * _NS
_L = 16
_CHUNK = _N_EDGES // _NW
_UNROLL = 5
_EWIN = _CHUNK + 112

_HI = -65536

_DN_T = (((1,), (1,)), ((), ()))


def _pack_bf16_pair(p, q):
    pi = lax.bitcast_convert_type(p, jnp.int32)
    qi = lax.bitcast_convert_type(q, jnp.int32)
    hi = pi & _HI
    lo = lax.shift_right_logical(qi + 0x8000, 16)
    return hi | lo


def _table_body(zb_ref, zp_ref, ab_ref, ap_ref, gb_ref, gp_ref, Wb_ref,
                Wp_ref, ob_ref, op_ref):
    ab = ab_ref[...]
    ap = ap_ref[...]
    Ab = jnp.concatenate([ab[:, :_GAT_DIM], ab[:, _GAT_DIM:]], axis=0)
    Ap = jnp.concatenate([ap[:, :_GAT_DIM], ap[:, _GAT_DIM:]], axis=0)
    Vb = jnp.dot(Ab, Wb_ref[...], preferred_element_type=jnp.float32)
    Vp = jnp.dot(Ap, Wp_ref[...], preferred_element_type=jnp.float32)
    Tb = lax.dot_general(Vb, zb_ref[...], _DN_T,
                         preferred_element_type=jnp.float32)
    Tp = lax.dot_general(Vp, zp_ref[...], _DN_T,
                         preferred_element_type=jnp.float32)
    e0 = jnp.exp(gb_ref[0])
    e1 = jnp.exp(gp_ref[0])
    w0 = e0 / (e0 + e1)
    w1 = e1 / (e0 + e1)
    ob_ref[...] = _pack_bf16_pair(Tb[0:1, :] * w0, Tb[1:2, :] * w0)
    op_ref[...] = _pack_bf16_pair(Tp[0:1, :] * w1, Tp[1:2, :] * w1)


_table_call = pl.pallas_call(
    _table_body,
    out_shape=(jax.ShapeDtypeStruct((1, _N_NODES), jnp.int32),
               jax.ShapeDtypeStruct((1, _N_NODES), jnp.int32)),
)


def _unpack(w):
    p = plsc.bitcast(w, jnp.float32)
    q = plsc.bitcast(lax.shift_left(w, 16), jnp.float32)
    return p, q


def _edge_body(tb_hbm, tp_hbm, ei_hbm, out_hbm, tb_v, tp_v, ed_v, out_v):
    wid = lax.axis_index("s") * _NC + lax.axis_index("c")
    base = wid * _CHUNK
    astart = pl.multiple_of(base - (base % 128), 128)
    pad = base % 128

    pltpu.sync_copy(tb_hbm.at[0], tb_v)
    pltpu.sync_copy(tp_hbm.at[0], tp_v)
    pltpu.sync_copy(ei_hbm.at[:, pl.ds(astart, _EWIN)], ed_v)

    @plsc.parallel_loop(0, _CHUNK, _L, unroll=_UNROLL)
    def step(off):
        s = ed_v[0, pl.ds(pad + off, _L)]
        d = ed_v[1, pl.ds(pad + off, _L)]
        pb_s, qb_s = _unpack(plsc.load_gather(tb_v, [s]))
        pb_d, qb_d = _unpack(plsc.load_gather(tb_v, [d]))
        pp_s, qp_s = _unpack(plsc.load_gather(tp_v, [s]))
        pp_d, qp_d = _unpack(plsc.load_gather(tp_v, [d]))
        x1b = pb_s + qb_d
        x2b = pb_d + qb_s
        x1p = pp_s + qp_d
        x2p = pp_d + qp_s
        lin = (x1b + x2b) + (x1p + x2p)
        mag = (jnp.abs(x1b) + jnp.abs(x2b)) + (jnp.abs(x1p) + jnp.abs(x2p))
        sc = 0.3 * lin + 0.2 * mag
        out_v[pl.ds(off, _L)] = 1.0 / (1.0 + jnp.exp(-sc))

    pltpu.sync_copy(out_v, out_hbm.at[pl.ds(base, _CHUNK)])


@functools.cache
def _edge_call():
  return functools.partial(
    pl.kernel,
    mesh=plsc.VectorSubcoreMesh(core_axis_name="c", subcore_axis_name="s"),
    out_type=jax.ShapeDtypeStruct((_N_EDGES,), jnp.float32),
    scratch_types=[
        pltpu.VMEM((_N_NODES,), jnp.int32),
        pltpu.VMEM((_N_NODES,), jnp.int32),
        pltpu.VMEM((2, _EWIN), jnp.int32),
        pltpu.VMEM((_CHUNK,), jnp.float32),
    ],
    compiler_params=pltpu.CompilerParams(needs_layout_passes=False),
  )(_edge_body)


def kernel(z_beh, z_pref, edge_index_undirected, W_b, W_p, a_b, a_p,
           gamma_b, gamma_p):
    tb, tp = _table_call(z_beh, z_pref, a_b.reshape(1, -1), a_p.reshape(1, -1),
                         gamma_b.reshape(1), gamma_p.reshape(1),
                         W_b, W_p)
    return _edge_call()(tb, tp, edge_index_undirected)


# speedup vs baseline: 1.1435x; 1.0399x over previous
import functools

import jax
import jax.numpy as jnp
from jax import lax
from jax.experimental import pallas as pl
from jax.experimental.pallas import tpu as pltpu
from jax.experimental.pallas import tpu_sc as plsc

_N_NODES = 10000
_N_EDGES = 320000
_PROJ_DIM = 128
_GAT_DIM = 64

_NC = 2
_NS = 16
_NW = _NC * _NS
_L = 16
_CHUNK = _N_EDGES // _NW
_UNROLL = 5
_EWIN = _CHUNK + 112

_HI = -65536

_DN_T = (((1,), (1,)), ((), ()))


def _pack_bf16_pair(p, q):
    pi = lax.bitcast_convert_type(p, jnp.int32)
    qi = lax.bitcast_convert_type(q, jnp.int32)
    hi = pi & _HI
    lo = lax.shift_right_logical(qi + 0x8000, 16)
    return hi | lo


def _table_body(zb_ref, zp_ref, ab_ref, ap_ref, gb_ref, gp_ref, Wb_ref,
                Wp_ref, ob_ref, op_ref):
    ab = ab_ref[...]
    ap = ap_ref[...]
    Ab = jnp.concatenate([ab[:, :_GAT_DIM], ab[:, _GAT_DIM:]], axis=0)
    Ap = jnp.concatenate([ap[:, :_GAT_DIM], ap[:, _GAT_DIM:]], axis=0)
    Vb = jnp.dot(Ab, Wb_ref[...], preferred_element_type=jnp.float32)
    Vp = jnp.dot(Ap, Wp_ref[...], preferred_element_type=jnp.float32)
    Tb = lax.dot_general(Vb, zb_ref[...], _DN_T,
                         preferred_element_type=jnp.float32)
    Tp = lax.dot_general(Vp, zp_ref[...], _DN_T,
                         preferred_element_type=jnp.float32)
    e0 = jnp.exp(gb_ref[0])
    e1 = jnp.exp(gp_ref[0])
    w0 = e0 / (e0 + e1)
    w1 = e1 / (e0 + e1)
    ob_ref[...] = _pack_bf16_pair(Tb[0:1, :] * w0, Tb[1:2, :] * w0)
    op_ref[...] = _pack_bf16_pair(Tp[0:1, :] * w1, Tp[1:2, :] * w1)


_table_call = pl.pallas_call(
    _table_body,
    out_shape=(jax.ShapeDtypeStruct((1, _N_NODES), jnp.int32),
               jax.ShapeDtypeStruct((1, _N_NODES), jnp.int32)),
)


def _unpack(w):
    p = plsc.bitcast(w, jnp.float32)
    q = plsc.bitcast(lax.shift_left(w, 16), jnp.float32)
    return p, q


def _edge_body(tb_hbm, tp_hbm, ei_hbm, out_hbm, tb_v, tp_v, ed_v, out_v, sem):
    wid = lax.axis_index("s") * _NC + lax.axis_index("c")
    base = wid * _CHUNK
    astart = pl.multiple_of(base - (base % 128), 128)
    pad = base % 128

    c1 = pltpu.async_copy(tb_hbm.at[0], tb_v, sem)
    c2 = pltpu.async_copy(tp_hbm.at[0], tp_v, sem)
    c3 = pltpu.async_copy(ei_hbm.at[:, pl.ds(astart, _EWIN)], ed_v, sem)
    c1.wait()
    c2.wait()
    c3.wait()

    @plsc.parallel_loop(0, _CHUNK, _L, unroll=_UNROLL)
    def step(off):
        s = ed_v[0, pl.ds(pad + off, _L)]
        d = ed_v[1, pl.ds(pad + off, _L)]
        pb_s, qb_s = _unpack(plsc.load_gather(tb_v, [s]))
        pb_d, qb_d = _unpack(plsc.load_gather(tb_v, [d]))
        pp_s, qp_s = _unpack(plsc.load_gather(tp_v, [s]))
        pp_d, qp_d = _unpack(plsc.load_gather(tp_v, [d]))
        x1b = pb_s + qb_d
        x2b = pb_d + qb_s
        x1p = pp_s + qp_d
        x2p = pp_d + qp_s
        lin = (x1b + x2b) + (x1p + x2p)
        mag = (jnp.abs(x1b) + jnp.abs(x2b)) + (jnp.abs(x1p) + jnp.abs(x2p))
        sc = 0.3 * lin + 0.2 * mag
        out_v[pl.ds(off, _L)] = 1.0 / (1.0 + jnp.exp(-sc))

    pltpu.sync_copy(out_v, out_hbm.at[pl.ds(base, _CHUNK)])


@functools.cache
def _edge_call():
  return functools.partial(
    pl.kernel,
    mesh=plsc.VectorSubcoreMesh(core_axis_name="c", subcore_axis_name="s"),
    out_type=jax.ShapeDtypeStruct((_N_EDGES,), jnp.float32),
    scratch_types=[
        pltpu.VMEM((_N_NODES,), jnp.int32),
        pltpu.VMEM((_N_NODES,), jnp.int32),
        pltpu.VMEM((2, _EWIN), jnp.int32),
        pltpu.VMEM((_CHUNK,), jnp.float32),
        pltpu.SemaphoreType.DMA,
    ],
    compiler_params=pltpu.CompilerParams(needs_layout_passes=False),
  )(_edge_body)


def kernel(z_beh, z_pref, edge_index_undirected, W_b, W_p, a_b, a_p,
           gamma_b, gamma_p):
    tb, tp = _table_call(z_beh, z_pref, a_b.reshape(1, -1), a_p.reshape(1, -1),
                         gamma_b.reshape(1), gamma_p.reshape(1),
                         W_b, W_p)
    return _edge_call()(tb, tp, edge_index_undirected)


# speedup vs baseline: 1.1663x; 1.0199x over previous
import functools

import jax
import jax.numpy as jnp
from jax import lax
from jax.experimental import pallas as pl
from jax.experimental.pallas import tpu as pltpu
from jax.experimental.pallas import tpu_sc as plsc

_N_NODES = 10000
_N_EDGES = 320000
_PROJ_DIM = 128
_GAT_DIM = 64

_NC = 2
_NS = 16
_NW = _NC * _NS
_L = 16
_CHUNK = _N_EDGES // _NW
_UNROLL = 5
_EWIN = _CHUNK + 112

_HI = -65536

_DN_T = (((1,), (1,)), ((), ()))


def _pack_bf16_pair(p, q):
    pi = lax.bitcast_convert_type(p, jnp.int32)
    qi = lax.bitcast_convert_type(q, jnp.int32)
    hi = pi & _HI
    lo = lax.shift_right_logical(qi + 0x8000, 16)
    return hi | lo


def _table_body(zb_ref, zp_ref, ab_ref, ap_ref, gb_ref, gp_ref, Wb_ref,
                Wp_ref, ob_ref, op_ref):
    ab = ab_ref[...]
    ap = ap_ref[...]
    Ab = jnp.concatenate([ab[:, :_GAT_DIM], ab[:, _GAT_DIM:]], axis=0)
    Ap = jnp.concatenate([ap[:, :_GAT_DIM], ap[:, _GAT_DIM:]], axis=0)
    Vb = jnp.dot(Ab, Wb_ref[...], preferred_element_type=jnp.float32)
    Vp = jnp.dot(Ap, Wp_ref[...], preferred_element_type=jnp.float32)
    Tb = lax.dot_general(Vb, zb_ref[...], _DN_T,
                         preferred_element_type=jnp.float32)
    Tp = lax.dot_general(Vp, zp_ref[...], _DN_T,
                         preferred_element_type=jnp.float32)
    e0 = jnp.exp(gb_ref[0])
    e1 = jnp.exp(gp_ref[0])
    w0 = e0 / (e0 + e1)
    w1 = e1 / (e0 + e1)
    ob_ref[...] = _pack_bf16_pair((Tb[0:1, :] + Tb[1:2, :]) * w0,
                                  (Tb[0:1, :] - Tb[1:2, :]) * w0)
    op_ref[...] = _pack_bf16_pair((Tp[0:1, :] + Tp[1:2, :]) * w1,
                                  (Tp[0:1, :] - Tp[1:2, :]) * w1)


_table_call = pl.pallas_call(
    _table_body,
    out_shape=(jax.ShapeDtypeStruct((1, _N_NODES), jnp.int32),
               jax.ShapeDtypeStruct((1, _N_NODES), jnp.int32)),
)


def _unpack(w):
    p = plsc.bitcast(w, jnp.float32)
    q = plsc.bitcast(lax.shift_left(w, 16), jnp.float32)
    return p, q


def _edge_body(tb_hbm, tp_hbm, ei_hbm, out_hbm, tb_v, tp_v, ed_v, out_v, sem):
    wid = lax.axis_index("s") * _NC + lax.axis_index("c")
    base = wid * _CHUNK
    astart = pl.multiple_of(base - (base % 128), 128)
    pad = base % 128

    c1 = pltpu.async_copy(tb_hbm.at[0], tb_v, sem)
    c2 = pltpu.async_copy(tp_hbm.at[0], tp_v, sem)
    c3 = pltpu.async_copy(ei_hbm.at[:, pl.ds(astart, _EWIN)], ed_v, sem)
    c1.wait()
    c2.wait()
    c3.wait()

    @plsc.parallel_loop(0, _CHUNK, _L, unroll=_UNROLL)
    def step(off):
        s = ed_v[0, pl.ds(pad + off, _L)]
        d = ed_v[1, pl.ds(pad + off, _L)]
        Sb_s, Db_s = _unpack(plsc.load_gather(tb_v, [s]))
        Sb_d, Db_d = _unpack(plsc.load_gather(tb_v, [d]))
        Sp_s, Dp_s = _unpack(plsc.load_gather(tp_v, [s]))
        Sp_d, Dp_d = _unpack(plsc.load_gather(tp_v, [d]))
        Sb = Sb_s + Sb_d
        Db = Db_s - Db_d
        Sp = Sp_s + Sp_d
        Dp = Dp_s - Dp_d
        lin = Sb + Sp
        mag = (jnp.maximum(jnp.abs(Sb), jnp.abs(Db))
               + jnp.maximum(jnp.abs(Sp), jnp.abs(Dp)))
        sc = 0.3 * lin + 0.2 * mag
        out_v[pl.ds(off, _L)] = 1.0 / (1.0 + jnp.exp(-sc))

    pltpu.sync_copy(out_v, out_hbm.at[pl.ds(base, _CHUNK)])


@functools.cache
def _edge_call():
  return functools.partial(
    pl.kernel,
    mesh=plsc.VectorSubcoreMesh(core_axis_name="c", subcore_axis_name="s"),
    out_type=jax.ShapeDtypeStruct((_N_EDGES,), jnp.float32),
    scratch_types=[
        pltpu.VMEM((_N_NODES,), jnp.int32),
        pltpu.VMEM((_N_NODES,), jnp.int32),
        pltpu.VMEM((2, _EWIN), jnp.int32),
        pltpu.VMEM((_CHUNK,), jnp.float32),
        pltpu.SemaphoreType.DMA,
    ],
    compiler_params=pltpu.CompilerParams(needs_layout_passes=False),
  )(_edge_body)


def kernel(z_beh, z_pref, edge_index_undirected, W_b, W_p, a_b, a_p,
           gamma_b, gamma_p):
    tb, tp = _table_call(z_beh, z_pref, a_b.reshape(1, -1), a_p.reshape(1, -1),
                         gamma_b.reshape(1), gamma_p.reshape(1),
                         W_b, W_p)
    return _edge_call()(tb, tp, edge_index_undirected)
